# baseline (speedup 1.0000x reference)
import functools

import jax
import jax.numpy as jnp
from jax import lax
from jax.experimental import pallas as pl
from jax.experimental.pallas import tpu as pltpu
from jax.experimental.pallas import tpu_sc as plsc

B = 16384
D = 64
C = 20
N = 100
RBLK = 128
NVREG = D // 16
DEPTH = 4


@functools.lru_cache(maxsize=None)
def _build_sc_kernel():
  info = plsc.get_sparse_core_info()
  nc, ns = info.num_cores, info.num_subcores
  nw = nc * ns
  rpw = B // nw
  nblk = rpw // RBLK
  mesh = plsc.VectorSubcoreMesh(core_axis_name="c", subcore_axis_name="s")

  scratch = (
      pltpu.VMEM((RBLK,), jnp.int32),
      pltpu.VMEM((RBLK, C), jnp.int32),
      pltpu.VMEM((RBLK, N), jnp.int32),
      pltpu.VMEM((RBLK, D), jnp.float32),
      [pltpu.VMEM((C, D), jnp.float32) for _ in range(DEPTH)],
      [pltpu.VMEM((N, D), jnp.float32) for _ in range(DEPTH)],
      pltpu.VMEM((RBLK, D), jnp.float32),
      pltpu.VMEM((RBLK, D), jnp.float32),
      [pltpu.SemaphoreType.DMA for _ in range(DEPTH)],
      pltpu.SemaphoreType.DMA,
  )

  @functools.partial(
      pl.kernel,
      out_type=(
          jax.ShapeDtypeStruct((B, D), jnp.float32),
          jax.ShapeDtypeStruct((B, D), jnp.float32),
          jax.ShapeDtypeStruct((B, D), jnp.float32),
      ),
      mesh=mesh,
      compiler_params=pltpu.CompilerParams(use_tc_tiling_on_sc=False),
      scratch_types=scratch,
  )
  def sc_kernel(center_hbm, ctx_hbm, neg_hbm, in_emb_hbm, out_emb_hbm,
                crows_o, ctxsum_o, negsum_o,
                cidx_v, ctxidx_v, negidx_v, crows_v, bufs_c, bufs_n,
                ctxsum_v, negsum_v, sems, sem_c):
    wid = lax.axis_index("s") * nc + lax.axis_index("c")

    def issue(b, slot):
      pltpu.make_async_copy(
          out_emb_hbm.at[ctxidx_v.at[b]], bufs_c[slot], sems[slot]).start()
      pltpu.make_async_copy(
          out_emb_hbm.at[negidx_v.at[b]], bufs_n[slot], sems[slot]).start()

    def drain(slot):
      pltpu.make_async_copy(
          out_emb_hbm.at[ctxidx_v.at[0]], bufs_c[slot], sems[slot]).wait()
      pltpu.make_async_copy(
          out_emb_hbm.at[negidx_v.at[0]], bufs_n[slot], sems[slot]).wait()

    def reduce_store(slot, b):
      for k in range(NVREG):
        sl = pl.ds(16 * k, 16)
        acc_c = bufs_c[slot][0, sl]
        for j in range(1, C):
          acc_c = acc_c + bufs_c[slot][j, sl]
        acc_n = bufs_n[slot][0, sl]
        for j in range(1, N):
          acc_n = acc_n + bufs_n[slot][j, sl]
        ctxsum_v[b, sl] = acc_c
        negsum_v[b, sl] = acc_n

    def block(blk, carry):
      base = pl.multiple_of(wid * rpw + blk * RBLK, RBLK)
      pltpu.sync_copy(center_hbm.at[pl.ds(base, RBLK)], cidx_v)
      pltpu.sync_copy(ctx_hbm.at[pl.ds(base, RBLK), :], ctxidx_v)
      pltpu.sync_copy(neg_hbm.at[pl.ds(base, RBLK), :], negidx_v)
      crows_cp = pltpu.make_async_copy(in_emb_hbm.at[cidx_v], crows_v, sem_c)
      crows_cp.start()

      for s in range(DEPTH - 1):
        issue(s, s)

      def group(g, c2):
        for s in range(DEPTH):
          b = g * DEPTH + s

          @pl.when(b + DEPTH - 1 < RBLK)
          def _():
            issue(b + DEPTH - 1, (s + DEPTH - 1) % DEPTH)

          drain(s)
          reduce_store(s, b)
        return c2

      lax.fori_loop(0, RBLK // DEPTH, group, 0)

      crows_cp.wait()
      pltpu.sync_copy(crows_v, crows_o.at[pl.ds(base, RBLK), :])
      pltpu.sync_copy(ctxsum_v, ctxsum_o.at[pl.ds(base, RBLK), :])
      pltpu.sync_copy(negsum_v, negsum_o.at[pl.ds(base, RBLK), :])
      return carry

    lax.fori_loop(0, nblk, block, 0)

  return sc_kernel


def _tc_score(crows, ctxsum, negsum):
  bt = 2048

  def body(c_ref, cs_ref, ns_ref, o_ref):
    c = c_ref[...]
    s_ctx = jnp.sum(cs_ref[...] * c, axis=1)
    s_neg = jnp.sum(ns_ref[...] * c, axis=1)
    o_ref[...] = -(jax.nn.log_sigmoid(s_ctx) + jax.nn.log_sigmoid(-s_neg))

  return pl.pallas_call(
      body,
      grid=(B // bt,),
      in_specs=[pl.BlockSpec((bt, D), lambda i: (i, 0))] * 3,
      out_specs=pl.BlockSpec((bt,), lambda i: (i,)),
      out_shape=jax.ShapeDtypeStruct((B,), jnp.float32),
  )(crows, ctxsum, negsum)


def kernel(center_word_label, context_words_labels, neg_words_labels,
           input_embedding, output_embedding):
  crows, ctxsum, negsum = _build_sc_kernel()(
      center_word_label.astype(jnp.int32),
      context_words_labels.astype(jnp.int32),
      neg_words_labels.astype(jnp.int32),
      input_embedding, output_embedding)
  return _tc_score(crows, ctxsum, negsum)


# speedup vs baseline: 1.0031x; 1.0031x over previous
import functools

import jax
import jax.numpy as jnp
from jax import lax
from jax.experimental import pallas as pl
from jax.experimental.pallas import tpu as pltpu
from jax.experimental.pallas import tpu_sc as plsc

B = 16384
D = 64
C = 20
N = 100
K = C + N
KP = 128
RBLK = 128
NVREG = D // 16
DEPTH = 4


@functools.lru_cache(maxsize=None)
def _build_sc_kernel():
  info = plsc.get_sparse_core_info()
  nc, ns = info.num_cores, info.num_subcores
  nw = nc * ns
  rpw = B // nw
  nblk = rpw // RBLK
  mesh = plsc.VectorSubcoreMesh(core_axis_name="c", subcore_axis_name="s")

  scratch = (
      pltpu.VMEM((RBLK,), jnp.int32),
      pltpu.VMEM((RBLK, KP), jnp.int32),
      pltpu.VMEM((RBLK, D), jnp.float32),
      [pltpu.VMEM((K, D), jnp.float32) for _ in range(DEPTH)],
      pltpu.VMEM((RBLK, D), jnp.float32),
      pltpu.VMEM((RBLK, D), jnp.float32),
      [pltpu.SemaphoreType.DMA for _ in range(DEPTH)],
      pltpu.SemaphoreType.DMA,
  )

  @functools.partial(
      pl.kernel,
      out_type=(
          jax.ShapeDtypeStruct((B, D), jnp.float32),
          jax.ShapeDtypeStruct((B, D), jnp.float32),
          jax.ShapeDtypeStruct((B, D), jnp.float32),
      ),
      mesh=mesh,
      compiler_params=pltpu.CompilerParams(use_tc_tiling_on_sc=False),
      scratch_types=scratch,
  )
  def sc_kernel(center_hbm, idx_hbm, in_emb_hbm, out_emb_hbm,
                crows_o, ctxsum_o, negsum_o,
                cidx_v, idx_v, crows_v, bufs, ctxsum_v, negsum_v,
                sems, sem_c):
    wid = lax.axis_index("s") * nc + lax.axis_index("c")

    def issue(b, slot):
      pltpu.make_async_copy(
          out_emb_hbm.at[idx_v.at[b, pl.ds(0, K)]], bufs[slot],
          sems[slot]).start()

    def drain(slot):
      pltpu.make_async_copy(
          out_emb_hbm.at[idx_v.at[0, pl.ds(0, K)]], bufs[slot],
          sems[slot]).wait()

    def reduce_store(slot, b):
      buf = bufs[slot]
      for k in range(NVREG):
        sl = pl.ds(16 * k, 16)
        acc_c = buf[0, sl]
        for j in range(1, C):
          acc_c = acc_c + buf[j, sl]
        acc_n = buf[C, sl]
        for j in range(C + 1, K):
          acc_n = acc_n + buf[j, sl]
        ctxsum_v[b, sl] = acc_c
        negsum_v[b, sl] = acc_n

    def block(blk, carry):
      base = pl.multiple_of(wid * rpw + blk * RBLK, RBLK)
      pltpu.sync_copy(center_hbm.at[pl.ds(base, RBLK)], cidx_v)
      pltpu.sync_copy(idx_hbm.at[pl.ds(base, RBLK), :], idx_v)
      crows_cp = pltpu.make_async_copy(in_emb_hbm.at[cidx_v], crows_v, sem_c)
      crows_cp.start()

      for s in range(DEPTH - 1):
        issue(s, s)

      def group(g, c2):
        for s in range(DEPTH):
          b = g * DEPTH + s

          @pl.when(b + DEPTH - 1 < RBLK)
          def _():
            issue(b + DEPTH - 1, (s + DEPTH - 1) % DEPTH)

          drain(s)
          reduce_store(s, b)
        return c2

      lax.fori_loop(0, RBLK // DEPTH, group, 0)

      crows_cp.wait()
      pltpu.sync_copy(crows_v, crows_o.at[pl.ds(base, RBLK), :])
      pltpu.sync_copy(ctxsum_v, ctxsum_o.at[pl.ds(base, RBLK), :])
      pltpu.sync_copy(negsum_v, negsum_o.at[pl.ds(base, RBLK), :])
      return carry

    lax.fori_loop(0, nblk, block, 0)

  return sc_kernel


def _tc_score(crows, ctxsum, negsum):
  bt = 2048

  def body(c_ref, cs_ref, ns_ref, o_ref):
    c = c_ref[...]
    s_ctx = jnp.sum(cs_ref[...] * c, axis=1)
    s_neg = jnp.sum(ns_ref[...] * c, axis=1)
    o_ref[...] = -(jax.nn.log_sigmoid(s_ctx) + jax.nn.log_sigmoid(-s_neg))

  return pl.pallas_call(
      body,
      grid=(B // bt,),
      in_specs=[pl.BlockSpec((bt, D), lambda i: (i, 0))] * 3,
      out_specs=pl.BlockSpec((bt,), lambda i: (i,)),
      out_shape=jax.ShapeDtypeStruct((B,), jnp.float32),
  )(crows, ctxsum, negsum)


def kernel(center_word_label, context_words_labels, neg_words_labels,
           input_embedding, output_embedding):
  idx_all = jnp.concatenate(
      [context_words_labels.astype(jnp.int32),
       neg_words_labels.astype(jnp.int32),
       jnp.zeros((B, KP - K), jnp.int32)], axis=1)
  crows, ctxsum, negsum = _build_sc_kernel()(
      center_word_label.astype(jnp.int32), idx_all,
      input_embedding, output_embedding)
  return _tc_score(crows, ctxsum, negsum)
